# Initial kernel scaffold; baseline (speedup 1.0000x reference)
import jax
import jax.numpy as jnp
from jax.experimental import pallas as pl


def kernel(state, choice):
    raise NotImplementedError("write your pallas kernel here")


# speedup vs baseline: 3.8051x; 3.8051x over previous
import functools

import jax
import jax.numpy as jnp
from jax import lax
from jax.experimental import pallas as pl
from jax.experimental.pallas import tpu as pltpu
from jax.experimental.pallas import tpu_sc as plsc

B = 16384
M = 4096
NW = 32
RPW = B // NW
L = 16
GROUPS = RPW // L
IDX_ROWS = RPW // 128


def _copy_body(x_ref, o_ref):
    o_ref[...] = x_ref[...]


def _tc_copy(state):
    blk = 256
    return pl.pallas_call(
        _copy_body,
        out_shape=jax.ShapeDtypeStruct((B, M), jnp.float32),
        grid=(B // blk,),
        in_specs=[pl.BlockSpec((blk, M), lambda i: (i, 0))],
        out_specs=pl.BlockSpec((blk, M), lambda i: (i, 0)),
    )(state)


def _sc_scatter_body(out_hbm, choice_hbm, choice_v, idx_v, val_v, sem):
    cid = lax.axis_index("c")
    sid = lax.axis_index("s")
    wid = sid * 2 + cid
    base = wid * RPW
    pltpu.sync_copy(choice_hbm.at[pl.ds(base, RPW)], choice_v)
    for g in range(GROUPS):
        cvec = choice_v[pl.ds(g * L, L)]
        rows = (base + g * L) + lax.iota(jnp.int32, L)
        flat = rows * M + cvec
        idx_v[g // 8, pl.ds((g % 8) * L, L)] = flat
    for j in range(IDX_ROWS):
        pltpu.async_copy(out_hbm.at[idx_v.at[j]], val_v.at[j], sem).wait()
    for g in range(GROUPS):
        r, c = g // 8, (g % 8) * L
        val_v[r, pl.ds(c, L)] = val_v[r, pl.ds(c, L)] + 1.0
    for j in range(IDX_ROWS):
        pltpu.async_copy(val_v.at[j], out_hbm.at[idx_v.at[j]], sem).wait()


_sc_scatter = functools.partial(
    pl.kernel,
    mesh=plsc.VectorSubcoreMesh(
        core_axis_name="c", subcore_axis_name="s", num_cores=2, num_subcores=16
    ),
    scratch_types=[
        pltpu.VMEM((RPW,), jnp.int32),
        pltpu.VMEM((IDX_ROWS, 128), jnp.int32),
        pltpu.VMEM((IDX_ROWS, 128), jnp.float32),
        pltpu.SemaphoreType.DMA,
    ],
)(_sc_scatter_body)


def kernel(state, choice):
    choice32 = choice.astype(jnp.int32)
    out = _tc_copy(state)
    out_ref = jax.new_ref(out.reshape(B * M))
    _sc_scatter(out_ref, choice32)
    return jax.freeze(out_ref).reshape(B, M)


# speedup vs baseline: 12.0746x; 3.1733x over previous
import functools

import jax
import jax.numpy as jnp
from jax import lax
from jax.experimental import pallas as pl
from jax.experimental.pallas import tpu as pltpu
from jax.experimental.pallas import tpu_sc as plsc

B = 16384
M = 4096
NW = 32
RPW = B // NW
L = 16
CHROWS = 8
NCH = RPW // CHROWS
NBUF = 3


def _sc_body(state_hbm, choice_hbm, out_hbm, choice_v,
             b0, b1, b2, si0, si1, si2, so0, so1, so2):
    bufs = [b0, b1, b2]
    sin = [si0, si1, si2]
    sout = [so0, so1, so2]
    wid = lax.axis_index("s") * 2 + lax.axis_index("c")
    base = wid * RPW

    pltpu.sync_copy(choice_hbm.at[pl.ds(base, RPW)], choice_v.at[pl.ds(0, RPW)])

    lane = lax.iota(jnp.int32, L)
    row_idx = lane & (CHROWS - 1)
    mask = lane < CHROWS
    ones = jnp.full((L,), 1.0, dtype=jnp.float32)

    h_in = {}
    h_out = {}
    for g in range(NBUF):
        h_in[g] = pltpu.async_copy(
            state_hbm.at[pl.ds(base + g * CHROWS, CHROWS)], bufs[g], sin[g])
    for g in range(NCH):
        b = g % NBUF
        if g >= 2:
            h_out[g - 2].wait()
            nxt = g + 1
            if nxt < NCH and nxt >= NBUF:
                h_in[nxt] = pltpu.async_copy(
                    state_hbm.at[pl.ds(base + nxt * CHROWS, CHROWS)],
                    bufs[nxt % NBUF], sin[nxt % NBUF])
        h_in[g].wait()
        cvec = choice_v[pl.ds(g * CHROWS, L)] & (M - 1)
        plsc.addupdate_scatter(bufs[b], [row_idx, cvec], ones, mask=mask)
        h_out[g] = pltpu.async_copy(
            bufs[b], out_hbm.at[pl.ds(base + g * CHROWS, CHROWS)], sout[b])
    h_out[NCH - 2].wait()
    h_out[NCH - 1].wait()


_sc_kernel = functools.partial(
    pl.kernel,
    out_type=jax.ShapeDtypeStruct((B, M), jnp.float32),
    mesh=plsc.VectorSubcoreMesh(
        core_axis_name="c", subcore_axis_name="s", num_cores=2, num_subcores=16
    ),
    compiler_params=pltpu.CompilerParams(needs_layout_passes=False),
    scratch_types=[
        pltpu.VMEM((RPW + CHROWS,), jnp.int32),
        pltpu.VMEM((CHROWS, M), jnp.float32),
        pltpu.VMEM((CHROWS, M), jnp.float32),
        pltpu.VMEM((CHROWS, M), jnp.float32),
        pltpu.SemaphoreType.DMA,
        pltpu.SemaphoreType.DMA,
        pltpu.SemaphoreType.DMA,
        pltpu.SemaphoreType.DMA,
        pltpu.SemaphoreType.DMA,
        pltpu.SemaphoreType.DMA,
    ],
)(_sc_body)


def kernel(state, choice):
    return _sc_kernel(state, choice.astype(jnp.int32))
